# baseline (speedup 1.0000x reference)
import functools

import jax
import jax.numpy as jnp
from jax import lax
from jax.experimental import pallas as pl
from jax.experimental.pallas import tpu as pltpu
from jax.experimental.pallas import tpu_sc as plsc

N = 10000
E = 320000
D = 128
H = 128
DE = 16

NW = 32
NCHUNK = 2560
EPAD = NCHUNK * 128
NPAD = 10112
ROWS_PT = NPAD // 16

CH_SLOW = 80
CH_FAST = 80
SLOW_C = 1
FAST_BASE = 16 * CH_SLOW
NB = 4

_EBS = 2560


def _drain(hbm_ref, buf, sem):
    pltpu.make_async_copy(hbm_ref.at[pl.ds(0, 128)], buf, sem).wait()


def _sc_segment(P_pad, src3, dst3, zeros_tile, with_cnt):
    mesh = plsc.VectorSubcoreMesh(core_axis_name="c", subcore_axis_name="s")
    out_type = [jax.ShapeDtypeStruct((2, NPAD, H), jnp.float32)]
    if with_cnt:
        out_type.append(jax.ShapeDtypeStruct((2, NPAD, H), jnp.float32))

    @functools.partial(
        pl.kernel,
        out_type=tuple(out_type) if with_cnt else out_type[0],
        mesh=mesh,
        scratch_types=[
            pltpu.VMEM((CH_FAST // 2, 128), jnp.int32),
            pltpu.VMEM((CH_FAST // 2, 128), jnp.int32),
            pltpu.VMEM((2, 128, H), jnp.float32),
            pltpu.VMEM_SHARED((NPAD, H), jnp.float32),
            pltpu.SemaphoreType.DMA,
            pltpu.SemaphoreType.DMA,
        ],
    )
    def seg_kernel(p_hbm, src_hbm, dst_hbm, z_hbm, ones_hbm, *rest):
        no = 2 if with_cnt else 1
        outs = rest[:no]
        src_v, dst_v, buf2, acc, gsem, wsem = rest[no:]
        c = lax.axis_index("c")
        s = lax.axis_index("s")
        pltpu.sync_copy(z_hbm, acc.at[pl.ds(s * ROWS_PT, ROWS_PT)])
        plsc.subcore_barrier()

        def run_main(start, hh):
            for h in range(2):
                pltpu.sync_copy(
                    src_hbm.at[pl.ds(start + h * hh, CH_FAST // 2)], src_v)
                pltpu.sync_copy(
                    dst_hbm.at[pl.ds(start + h * hh, CH_FAST // 2)], dst_v)
                def body(j, carry):
                    pltpu.sync_copy(p_hbm.at[src_v.at[j]], buf2.at[0])
                    pltpu.sync_copy(buf2.at[0], acc.at[dst_v.at[j]],
                                    add=True)
                    return carry

                lax.fori_loop(0, hh, body, 0)

        @pl.when(c == SLOW_C)
        def _():
            run_main(s * CH_SLOW, CH_SLOW // 2)

        @pl.when(c != SLOW_C)
        def _():
            run_main(FAST_BASE + s * CH_FAST, CH_FAST // 2)

        plsc.subcore_barrier()
        pltpu.sync_copy(acc.at[pl.ds(s * ROWS_PT, ROWS_PT)],
                        outs[0].at[c, pl.ds(s * ROWS_PT, ROWS_PT)])
        if with_cnt:
            plsc.subcore_barrier()
            pltpu.sync_copy(z_hbm, acc.at[pl.ds(s * ROWS_PT, ROWS_PT)])
            pltpu.sync_copy(ones_hbm, buf2.at[0])
            plsc.subcore_barrier()

            def run_cnt(start, hh):
                for h in range(2):
                    pltpu.sync_copy(
                        dst_hbm.at[pl.ds(start + h * hh, CH_FAST // 2)],
                        dst_v)

                    @pl.loop(0, hh, step=8)
                    def _(j0):
                        descs = [
                            pltpu.async_copy(buf2.at[0],
                                             acc.at[dst_v.at[j0 + u]],
                                             wsem, add=True)
                            for u in range(8)
                        ]
                        for d in descs:
                            d.wait()

            @pl.when(c == SLOW_C)
            def _():
                run_cnt(s * CH_SLOW, CH_SLOW // 2)

            @pl.when(c != SLOW_C)
            def _():
                run_cnt(FAST_BASE + s * CH_FAST, CH_FAST // 2)

            plsc.subcore_barrier()
            pltpu.sync_copy(acc.at[pl.ds(s * ROWS_PT, ROWS_PT)],
                            outs[1].at[c, pl.ds(s * ROWS_PT, ROWS_PT)])

    ones128 = jnp.ones((128, H), jnp.float32)
    return seg_kernel(P_pad, src3, dst3, zeros_tile, ones128)


def _sc_gather2(A_pad, B_pad, src3, dst3):
    mesh = plsc.VectorSubcoreMesh(core_axis_name="c", subcore_axis_name="s")

    @functools.partial(
        pl.kernel,
        out_type=(jax.ShapeDtypeStruct((EPAD, H), jnp.float32),
                  jax.ShapeDtypeStruct((EPAD, H), jnp.float32)),
        mesh=mesh,
        scratch_types=[
            pltpu.VMEM((CH_FAST, 128), jnp.int32),
            pltpu.VMEM((CH_FAST, 128), jnp.int32),
        ] + [pltpu.VMEM((128, H), jnp.float32) for _ in range(NB)]
          + [pltpu.SemaphoreType.DMA for _ in range(2 * NB)],
    )
    def gather_kernel(a_hbm, b_hbm, src_hbm, dst_hbm, g1_hbm, g2_hbm, *rest):
        src_v, dst_v = rest[:2]
        bufs = rest[2:2 + NB]
        gsems = rest[2 + NB:2 + 2 * NB]
        wsems = rest[2 + 2 * NB:2 + 3 * NB]
        c = lax.axis_index("c")
        s = lax.axis_index("s")

        tabs = (a_hbm, b_hbm)
        idxs = (src_v, dst_v)
        g_outs = (g1_hbm, g2_hbm)

        def run(start, nch):
            pltpu.sync_copy(src_hbm.at[pl.ds(start, CH_FAST)], src_v)
            pltpu.sync_copy(dst_hbm.at[pl.ds(start, CH_FAST)], dst_v)
            base = start * 128

            def body(j, carry):
                for st in range(2):
                    pltpu.sync_copy(tabs[st].at[idxs[st].at[j]], bufs[st])
                    pltpu.sync_copy(
                        bufs[st],
                        g_outs[st].at[pl.ds(base + j * 128, 128)])
                return carry

            lax.fori_loop(0, nch, body, 0)

        @pl.when(c == SLOW_C)
        def _():
            run(s * CH_SLOW, CH_SLOW)

        @pl.when(c != SLOW_C)
        def _():
            run(FAST_BASE + s * CH_FAST, CH_FAST)

    return gather_kernel(A_pad, B_pad, src3, dst3)


def _tc_pre_body(x_ref, wl0t_ref, wr0t_ref, bl0_ref, p0_ref, q0_ref):
    xv = x_ref[...]
    p0_ref[...] = jnp.dot(xv, wl0t_ref[...], preferred_element_type=jnp.float32)
    q0_ref[...] = jnp.dot(xv, wr0t_ref[...],
                          preferred_element_type=jnp.float32) + bl0_ref[...]


def _tc_pre(x, Wl0, Wr0, bl0):
    return pl.pallas_call(
        _tc_pre_body,
        out_shape=(jax.ShapeDtypeStruct((N, H), jnp.float32),
                   jax.ShapeDtypeStruct((N, H), jnp.float32)),
    )(x, Wl0.T, Wr0.T, bl0[None, :])


def _ln_relu(o, g, be):
    mu = jnp.mean(o, axis=-1, keepdims=True)
    var = jnp.mean((o - mu) ** 2, axis=-1, keepdims=True)
    return jnp.maximum((o - mu) * lax.rsqrt(var + 1e-5) * g + be, 0.0)


def _tc_mid_body(sa_ref, sb_ref, ca_ref, cb_ref, q0_ref, g0_ref, be0_ref,
                 wl1t_ref, wr1t_ref, bl1_ref, p1_ref, q1_ref, inv_ref):
    seg = sa_ref[...][:N, :] + sb_ref[...][:N, :]
    cnt = ca_ref[...][:N, :1] + cb_ref[...][:N, :1]
    inv = 1.0 / jnp.maximum(cnt, 1.0)
    h1 = _ln_relu(seg * inv + q0_ref[...], g0_ref[...], be0_ref[...])
    p1_ref[...] = jnp.dot(h1, wl1t_ref[...], preferred_element_type=jnp.float32)
    q1_ref[...] = jnp.dot(h1, wr1t_ref[...],
                          preferred_element_type=jnp.float32) + bl1_ref[...]
    inv_ref[...] = inv


def _tc_mid(S0, C0, Q0, g0, be0, Wl1, Wr1, bl1):
    return pl.pallas_call(
        _tc_mid_body,
        out_shape=(jax.ShapeDtypeStruct((N, H), jnp.float32),
                   jax.ShapeDtypeStruct((N, H), jnp.float32),
                   jax.ShapeDtypeStruct((N, 1), jnp.float32)),
    )(S0[0], S0[1], C0[0], C0[1], Q0, g0[None, :], be0[None, :], Wl1.T, Wr1.T,
      bl1[None, :])


def _tc_fin_body(sa_ref, sb_ref, q1_ref, inv_ref, g1_ref, be1_ref, w1at_ref,
                 w1bt_ref, b1_ref, a_ref, b_ref):
    seg = sa_ref[...][:N, :] + sb_ref[...][:N, :]
    h2 = _ln_relu(seg * inv_ref[...] + q1_ref[...], g1_ref[...], be1_ref[...])
    a_ref[...] = jnp.dot(h2, w1at_ref[...], preferred_element_type=jnp.float32)
    b_ref[...] = jnp.dot(h2, w1bt_ref[...],
                         preferred_element_type=jnp.float32) + b1_ref[...]


def _tc_fin(S1, Q1, inv, g1, be1, W1a, W1b, b1):
    return pl.pallas_call(
        _tc_fin_body,
        out_shape=(jax.ShapeDtypeStruct((N, H), jnp.float32),
                   jax.ShapeDtypeStruct((N, H), jnp.float32)),
    )(S1[0], S1[1], Q1, inv, g1[None, :], be1[None, :], W1a.T, W1b.T,
      b1[None, :])


def _edge_mlp_body(g1_ref, g2_ref, ea_ref, w1ct_ref, w2t_ref, b2_ref, w3t_ref,
                   b3_ref, out_ref):
    z1 = g1_ref[...] + g2_ref[...] + jnp.dot(
        ea_ref[...], w1ct_ref[...], preferred_element_type=jnp.float32)
    z1 = jnp.maximum(z1, 0.0)
    z2 = jnp.dot(z1, w2t_ref[...], preferred_element_type=jnp.float32)
    z2 = jnp.maximum(z2 + b2_ref[...], 0.0)
    z3 = jnp.dot(z2, w3t_ref[...], preferred_element_type=jnp.float32)
    out_ref[...] = z3 + b3_ref[...]


def _edge_mlp(G1, G2, ea, W1c, W2, b2, W3, b3):
    nblk = E // _EBS
    out = pl.pallas_call(
        _edge_mlp_body,
        grid=(nblk,),
        in_specs=[
            pl.BlockSpec((_EBS, H), lambda i: (i, 0)),
            pl.BlockSpec((_EBS, H), lambda i: (i, 0)),
            pl.BlockSpec((_EBS, DE), lambda i: (i, 0)),
            pl.BlockSpec((DE, H), lambda i: (0, 0)),
            pl.BlockSpec((H, H // 2), lambda i: (0, 0)),
            pl.BlockSpec((1, H // 2), lambda i: (0, 0)),
            pl.BlockSpec((H // 2, 1), lambda i: (0, 0)),
            pl.BlockSpec((1, 1), lambda i: (0, 0)),
        ],
        out_specs=pl.BlockSpec((_EBS, 1), lambda i: (i, 0)),
        out_shape=jax.ShapeDtypeStruct((E, 1), jnp.float32),
    )(G1, G2, ea, W1c.T, W2.T, b2[None, :], W3.T, b3[None, :])
    return out[:, 0]


def kernel(x, edge_index, edge_attr, Wl0, bl0, Wr0, g0, be0, Wl1, bl1, Wr1,
           g1, be1, W1, b1, W2, b2, W3, b3):
    src = edge_index[0]
    dst = edge_index[1]
    pad = EPAD - E
    src3 = jnp.concatenate(
        [src, jnp.full((pad,), N, jnp.int32)]).reshape(NCHUNK, 128)
    dst3 = jnp.concatenate(
        [dst, jnp.full((pad,), N, jnp.int32)]).reshape(NCHUNK, 128)
    zeros_h = jnp.zeros((ROWS_PT, H), jnp.float32)

    P0, Q0 = _tc_pre(x, Wl0, Wr0, bl0)
    P0p = jnp.pad(P0, ((0, NPAD - N), (0, 0)))
    S0, C0 = _sc_segment(P0p, src3, dst3, zeros_h, True)

    P1, Q1, inv = _tc_mid(S0, C0, Q0, g0, be0, Wl1, Wr1, bl1)
    P1p = jnp.pad(P1, ((0, NPAD - N), (0, 0)))
    S1 = _sc_segment(P1p, src3, dst3, zeros_h, False)

    A, B = _tc_fin(S1, Q1, inv, g1, be1, W1[:, :H], W1[:, H:2 * H], b1)
    Ap = jnp.pad(A, ((0, NPAD - N), (0, 0)))
    Bp = jnp.pad(B, ((0, NPAD - N), (0, 0)))
    G1, G2 = _sc_gather2(Ap, Bp, src3, dst3)

    return _edge_mlp(G1, G2, edge_attr, W1[:, 2 * H:], W2, b2, W3, b3)


# speedup vs baseline: 1.3745x; 1.3745x over previous
import functools

import jax
import jax.numpy as jnp
from jax import lax
from jax.experimental import pallas as pl
from jax.experimental.pallas import tpu as pltpu
from jax.experimental.pallas import tpu_sc as plsc

N = 10000
E = 320000
D = 128
H = 128
DE = 16

NW = 32
CH = 79
EPAD = NW * CH * 128
NPAD = 10112
ROWS_PT = NPAD // 16

_EBS = 2560


def _sc_segment(P_pad, src3, dst3, zeros_tile, with_cnt):
    mesh = plsc.VectorSubcoreMesh(core_axis_name="c", subcore_axis_name="s")
    out_type = [jax.ShapeDtypeStruct((2, NPAD, H), jnp.float32)]
    if with_cnt:
        out_type.append(jax.ShapeDtypeStruct((2, NPAD, H), jnp.float32))

    @functools.partial(
        pl.kernel,
        out_type=tuple(out_type) if with_cnt else out_type[0],
        mesh=mesh,
        scratch_types=[
            pltpu.VMEM((CH, 128), jnp.int32),
            pltpu.VMEM((CH, 128), jnp.int32),
            pltpu.VMEM((128, H), jnp.float32),
            pltpu.VMEM_SHARED((NPAD, H), jnp.float32),
        ],
    )
    def seg_kernel(p_hbm, src_hbm, dst_hbm, z_hbm, ones_hbm, *rest):
        no = 2 if with_cnt else 1
        outs = rest[:no]
        src_v, dst_v, buf, acc = rest[no:]
        c = lax.axis_index("c")
        s = lax.axis_index("s")
        wid = s * 2 + c
        pltpu.sync_copy(z_hbm, acc.at[pl.ds(s * ROWS_PT, ROWS_PT)])
        pltpu.sync_copy(src_hbm.at[wid], src_v)
        pltpu.sync_copy(dst_hbm.at[wid], dst_v)
        plsc.subcore_barrier()

        def body(j, carry):
            pltpu.sync_copy(p_hbm.at[src_v.at[j]], buf)
            pltpu.sync_copy(buf, acc.at[dst_v.at[j]], add=True)
            return carry

        lax.fori_loop(0, CH, body, 0)
        plsc.subcore_barrier()
        pltpu.sync_copy(acc.at[pl.ds(s * ROWS_PT, ROWS_PT)],
                        outs[0].at[c, pl.ds(s * ROWS_PT, ROWS_PT)])
        if with_cnt:
            plsc.subcore_barrier()
            pltpu.sync_copy(z_hbm, acc.at[pl.ds(s * ROWS_PT, ROWS_PT)])
            pltpu.sync_copy(ones_hbm, buf)
            plsc.subcore_barrier()

            def cnt_body(j, carry):
                pltpu.sync_copy(buf, acc.at[dst_v.at[j]], add=True)
                return carry

            lax.fori_loop(0, CH, cnt_body, 0)
            plsc.subcore_barrier()
            pltpu.sync_copy(acc.at[pl.ds(s * ROWS_PT, ROWS_PT)],
                            outs[1].at[c, pl.ds(s * ROWS_PT, ROWS_PT)])

    ones128 = jnp.ones((128, H), jnp.float32)
    return seg_kernel(P_pad, src3, dst3, zeros_tile, ones128)


def _sc_gather2(A_pad, B_pad, src3, dst3):
    mesh = plsc.VectorSubcoreMesh(core_axis_name="c", subcore_axis_name="s")

    @functools.partial(
        pl.kernel,
        out_type=(jax.ShapeDtypeStruct((EPAD, H), jnp.float32),
                  jax.ShapeDtypeStruct((EPAD, H), jnp.float32)),
        mesh=mesh,
        scratch_types=[
            pltpu.VMEM((CH, 128), jnp.int32),
            pltpu.VMEM((CH, 128), jnp.int32),
            pltpu.VMEM((128, H), jnp.float32),
            pltpu.VMEM((128, H), jnp.float32),
        ],
    )
    def gather_kernel(a_hbm, b_hbm, src_hbm, dst_hbm, g1_hbm, g2_hbm,
                      src_v, dst_v, buf_a, buf_b):
        c = lax.axis_index("c")
        s = lax.axis_index("s")
        wid = s * 2 + c
        base = wid * (CH * 128)
        pltpu.sync_copy(src_hbm.at[wid], src_v)
        pltpu.sync_copy(dst_hbm.at[wid], dst_v)

        def body(j, carry):
            pltpu.sync_copy(a_hbm.at[src_v.at[j]], buf_a)
            pltpu.sync_copy(buf_a, g1_hbm.at[pl.ds(base + j * 128, 128)])
            pltpu.sync_copy(b_hbm.at[dst_v.at[j]], buf_b)
            pltpu.sync_copy(buf_b, g2_hbm.at[pl.ds(base + j * 128, 128)])
            return carry

        lax.fori_loop(0, CH, body, 0)

    return gather_kernel(A_pad, B_pad, src3, dst3)


def _tc_pre_body(x_ref, wl0t_ref, wr0t_ref, bl0_ref, p0_ref, q0_ref):
    xv = x_ref[...]
    p0_ref[...] = jnp.dot(xv, wl0t_ref[...], preferred_element_type=jnp.float32)
    q0_ref[...] = jnp.dot(xv, wr0t_ref[...],
                          preferred_element_type=jnp.float32) + bl0_ref[...]


def _tc_pre(x, Wl0, Wr0, bl0):
    return pl.pallas_call(
        _tc_pre_body,
        out_shape=(jax.ShapeDtypeStruct((N, H), jnp.float32),
                   jax.ShapeDtypeStruct((N, H), jnp.float32)),
    )(x, Wl0.T, Wr0.T, bl0[None, :])


def _ln_relu(o, g, be):
    mu = jnp.mean(o, axis=-1, keepdims=True)
    var = jnp.mean((o - mu) ** 2, axis=-1, keepdims=True)
    return jnp.maximum((o - mu) * lax.rsqrt(var + 1e-5) * g + be, 0.0)


def _tc_mid_body(sa_ref, sb_ref, ca_ref, cb_ref, q0_ref, g0_ref, be0_ref,
                 wl1t_ref, wr1t_ref, bl1_ref, p1_ref, q1_ref, inv_ref):
    seg = sa_ref[...][:N, :] + sb_ref[...][:N, :]
    cnt = ca_ref[...][:N, :1] + cb_ref[...][:N, :1]
    inv = 1.0 / jnp.maximum(cnt, 1.0)
    h1 = _ln_relu(seg * inv + q0_ref[...], g0_ref[...], be0_ref[...])
    p1_ref[...] = jnp.dot(h1, wl1t_ref[...], preferred_element_type=jnp.float32)
    q1_ref[...] = jnp.dot(h1, wr1t_ref[...],
                          preferred_element_type=jnp.float32) + bl1_ref[...]
    inv_ref[...] = inv


def _tc_mid(S0, C0, Q0, g0, be0, Wl1, Wr1, bl1):
    return pl.pallas_call(
        _tc_mid_body,
        out_shape=(jax.ShapeDtypeStruct((N, H), jnp.float32),
                   jax.ShapeDtypeStruct((N, H), jnp.float32),
                   jax.ShapeDtypeStruct((N, 1), jnp.float32)),
    )(S0[0], S0[1], C0[0], C0[1], Q0, g0[None, :], be0[None, :], Wl1.T, Wr1.T,
      bl1[None, :])


def _tc_fin_body(sa_ref, sb_ref, q1_ref, inv_ref, g1_ref, be1_ref, w1at_ref,
                 w1bt_ref, b1_ref, a_ref, b_ref):
    seg = sa_ref[...][:N, :] + sb_ref[...][:N, :]
    h2 = _ln_relu(seg * inv_ref[...] + q1_ref[...], g1_ref[...], be1_ref[...])
    a_ref[...] = jnp.dot(h2, w1at_ref[...], preferred_element_type=jnp.float32)
    b_ref[...] = jnp.dot(h2, w1bt_ref[...],
                         preferred_element_type=jnp.float32) + b1_ref[...]


def _tc_fin(S1, Q1, inv, g1, be1, W1a, W1b, b1):
    return pl.pallas_call(
        _tc_fin_body,
        out_shape=(jax.ShapeDtypeStruct((N, H), jnp.float32),
                   jax.ShapeDtypeStruct((N, H), jnp.float32)),
    )(S1[0], S1[1], Q1, inv, g1[None, :], be1[None, :], W1a.T, W1b.T,
      b1[None, :])


def _edge_mlp_body(g1_ref, g2_ref, ea_ref, w1ct_ref, w2t_ref, b2_ref, w3t_ref,
                   b3_ref, out_ref):
    z1 = g1_ref[...] + g2_ref[...] + jnp.dot(
        ea_ref[...], w1ct_ref[...], preferred_element_type=jnp.float32)
    z1 = jnp.maximum(z1, 0.0)
    z2 = jnp.dot(z1, w2t_ref[...], preferred_element_type=jnp.float32)
    z2 = jnp.maximum(z2 + b2_ref[...], 0.0)
    z3 = jnp.dot(z2, w3t_ref[...], preferred_element_type=jnp.float32)
    out_ref[...] = z3 + b3_ref[...]


def _edge_mlp(G1, G2, ea, W1c, W2, b2, W3, b3):
    nblk = E // _EBS
    out = pl.pallas_call(
        _edge_mlp_body,
        grid=(nblk,),
        in_specs=[
            pl.BlockSpec((_EBS, H), lambda i: (i, 0)),
            pl.BlockSpec((_EBS, H), lambda i: (i, 0)),
            pl.BlockSpec((_EBS, DE), lambda i: (i, 0)),
            pl.BlockSpec((DE, H), lambda i: (0, 0)),
            pl.BlockSpec((H, H // 2), lambda i: (0, 0)),
            pl.BlockSpec((1, H // 2), lambda i: (0, 0)),
            pl.BlockSpec((H // 2, 1), lambda i: (0, 0)),
            pl.BlockSpec((1, 1), lambda i: (0, 0)),
        ],
        out_specs=pl.BlockSpec((_EBS, 1), lambda i: (i, 0)),
        out_shape=jax.ShapeDtypeStruct((E, 1), jnp.float32),
    )(G1, G2, ea, W1c.T, W2.T, b2[None, :], W3.T, b3[None, :])
    return out[:, 0]


def kernel(x, edge_index, edge_attr, Wl0, bl0, Wr0, g0, be0, Wl1, bl1, Wr1,
           g1, be1, W1, b1, W2, b2, W3, b3):
    src = edge_index[0]
    dst = edge_index[1]
    pad = EPAD - E
    src3 = jnp.concatenate(
        [src, jnp.full((pad,), N, jnp.int32)]).reshape(NW, CH, 128)
    dst3 = jnp.concatenate(
        [dst, jnp.full((pad,), N, jnp.int32)]).reshape(NW, CH, 128)
    zeros_h = jnp.zeros((ROWS_PT, H), jnp.float32)

    P0, Q0 = _tc_pre(x, Wl0, Wr0, bl0)
    P0p = jnp.pad(P0, ((0, NPAD - N), (0, 0)))
    S0, C0 = _sc_segment(P0p, src3, dst3, zeros_h, True)

    P1, Q1, inv = _tc_mid(S0, C0, Q0, g0, be0, Wl1, Wr1, bl1)
    P1p = jnp.pad(P1, ((0, NPAD - N), (0, 0)))
    S1 = _sc_segment(P1p, src3, dst3, zeros_h, False)

    A, B = _tc_fin(S1, Q1, inv, g1, be1, W1[:, :H], W1[:, H:2 * H], b1)
    Ap = jnp.pad(A, ((0, NPAD - N), (0, 0)))
    Bp = jnp.pad(B, ((0, NPAD - N), (0, 0)))
    G1, G2 = _sc_gather2(Ap, Bp, src3, dst3)

    return _edge_mlp(G1, G2, edge_attr, W1[:, 2 * H:], W2, b2, W3, b3)
